# Initial kernel scaffold; baseline (speedup 1.0000x reference)
import jax
import jax.numpy as jnp
from jax.experimental import pallas as pl


def kernel(inputs, emb_w, in_w, in_b, out_w, out_b):
    raise NotImplementedError("write your pallas kernel here")


# speedup vs baseline: 1.0908x; 1.0908x over previous
import functools

import jax
import jax.numpy as jnp
from jax import lax
from jax.experimental import pallas as pl
from jax.experimental.pallas import tpu as pltpu
from jax.experimental.pallas import tpu_sc as plsc

N_TOKENS = 16384
INPUT_DIM = 768
N_EMB = 8192
EMB_DIM = 256

TM = 512
TE = 2048

_NC = 2
_NS = 16
_NW = _NC * _NS
_B_PER_W = N_TOKENS // _NW
_CHUNK = 128


def _bf16(v):
    return v.astype(jnp.bfloat16)


def _argmin_body(x_ref, inw_ref, inb_ref, emb_ref, idx_ref,
                 flat_ref, a2_ref, bv_ref, bi_ref):
    e = pl.program_id(1)

    @pl.when(e == 0)
    def _():
        acc = lax.dot_general(
            _bf16(x_ref[:, 0:256]), _bf16(inw_ref[:, 0:256]),
            (((1,), (1,)), ((), ())), preferred_element_type=jnp.float32)
        for i in range(1, 3):
            acc = acc + lax.dot_general(
                _bf16(x_ref[:, i * 256:(i + 1) * 256]),
                _bf16(inw_ref[:, i * 256:(i + 1) * 256]),
                (((1,), (1,)), ((), ())), preferred_element_type=jnp.float32)
        flat = acc + inb_ref[...]
        a2_ref[...] = jnp.sum(flat * flat, axis=1, keepdims=True)
        flat_ref[...] = _bf16(flat).astype(jnp.float32)
        bv_ref[...] = jnp.full((TM, 1), jnp.inf, dtype=jnp.float32)
        bi_ref[...] = jnp.zeros((TM, 1), dtype=jnp.int32)

    emb = emb_ref[...]
    s = lax.dot_general(
        flat_ref[...], emb, (((1,), (1,)), ((), ())),
        preferred_element_type=jnp.float32)
    b2 = jnp.sum(emb * emb, axis=1)[None, :]
    d2 = (a2_ref[...] + b2) - 2.0 * s
    lv = jnp.min(d2, axis=1, keepdims=True)
    la = jnp.argmin(d2, axis=1).astype(jnp.int32)[:, None] + e * TE
    better = lv < bv_ref[...]
    bv_ref[...] = jnp.where(better, lv, bv_ref[...])
    bi_ref[...] = jnp.where(better, la, bi_ref[...])

    @pl.when(e == pl.num_programs(1) - 1)
    def _():
        idx_ref[...] = bi_ref[...]


def _encode_indices(inputs, in_w, in_b2d, emb_w):
    return pl.pallas_call(
        _argmin_body,
        grid=(N_TOKENS // TM, N_EMB // TE),
        in_specs=[
            pl.BlockSpec((TM, INPUT_DIM), lambda t, e: (t, 0)),
            pl.BlockSpec((EMB_DIM, INPUT_DIM), lambda t, e: (0, 0)),
            pl.BlockSpec((1, EMB_DIM), lambda t, e: (0, 0)),
            pl.BlockSpec((TE, EMB_DIM), lambda t, e: (e, 0)),
        ],
        out_specs=pl.BlockSpec((TM, 1), lambda t, e: (t, 0)),
        out_shape=jax.ShapeDtypeStruct((N_TOKENS, 1), jnp.int32),
        scratch_shapes=[
            pltpu.VMEM((TM, EMB_DIM), jnp.float32),
            pltpu.VMEM((TM, 1), jnp.float32),
            pltpu.VMEM((TM, 1), jnp.float32),
            pltpu.VMEM((TM, 1), jnp.int32),
        ],
        compiler_params=pltpu.CompilerParams(
            dimension_semantics=("parallel", "arbitrary")),
    )(inputs, in_w, in_b2d, emb_w)


def _table_body(emb_ref, outw_ref, outb_ref, tab_ref):
    tab_ref[...] = lax.dot_general(
        _bf16(emb_ref[...]).astype(jnp.float32), outw_ref[...],
        (((1,), (1,)), ((), ())),
        preferred_element_type=jnp.float32) + outb_ref[...]


def _make_table(emb_w, out_w, out_b2d):
    TB = 2048
    return pl.pallas_call(
        _table_body,
        grid=(N_EMB // TB,),
        in_specs=[
            pl.BlockSpec((TB, EMB_DIM), lambda i: (i, 0)),
            pl.BlockSpec((INPUT_DIM, EMB_DIM), lambda i: (0, 0)),
            pl.BlockSpec((1, INPUT_DIM), lambda i: (0, 0)),
        ],
        out_specs=pl.BlockSpec((TB, INPUT_DIM), lambda i: (i, 0)),
        out_shape=jax.ShapeDtypeStruct((N_EMB, INPUT_DIM), jnp.float32),
    )(emb_w, out_w, out_b2d)


@functools.cache
def _gather_rows_kernel():
    @functools.partial(
        pl.kernel,
        mesh=plsc.VectorSubcoreMesh(core_axis_name="c", subcore_axis_name="s"),
        out_type=jax.ShapeDtypeStruct((N_TOKENS, INPUT_DIM), jnp.float32),
        scratch_types=[
            pltpu.VMEM((_CHUNK,), jnp.int32),
            pltpu.VMEM((_CHUNK, INPUT_DIM), jnp.float32),
            pltpu.SemaphoreType.DMA,
        ],
    )
    def _gather_rows(table_hbm, idx_hbm, out_hbm, idx_v, rows_v, sem):
        wid = lax.axis_index("s") * _NC + lax.axis_index("c")
        base = wid * _B_PER_W
        for c in range(_B_PER_W // _CHUNK):
            off = base + c * _CHUNK
            pltpu.sync_copy(idx_hbm.at[pl.ds(off, _CHUNK)], idx_v)
            pltpu.async_copy(table_hbm.at[idx_v], rows_v, sem).wait()
            pltpu.sync_copy(rows_v, out_hbm.at[pl.ds(off, _CHUNK)])

    return _gather_rows


def kernel(inputs, emb_w, in_w, in_b, out_w, out_b):
    idx = _encode_indices(inputs, in_w, in_b.reshape(1, -1), emb_w)
    table = _make_table(emb_w, out_w, out_b.reshape(1, -1))
    return _gather_rows_kernel()(table, idx.reshape(-1))
